# Initial kernel scaffold; baseline (speedup 1.0000x reference)
import jax
import jax.numpy as jnp
from jax.experimental import pallas as pl


def kernel(x, edge_index, Wl1, Wr1, b1, g1, be1, Wl2, Wr2, b2, g2, be2, Wl3, Wr3, b3):
    raise NotImplementedError("write your pallas kernel here")


# speedup vs baseline: 4.3205x; 4.3205x over previous
import jax
import jax.numpy as jnp
from jax import lax
from jax.experimental import pallas as pl
from jax.experimental.pallas import tpu as pltpu
from jax.experimental.pallas import tpu_sc as plsc

NUM_NODES = 10000
NPAD = 10240
NC = 2
NS = 16
NW = NC * NS
CHUNK = 128
H = 128
BN = 1000

_CHPT = NPAD // CHUNK // NS
_ROWS = NPAD // NS


def _sc_mesh():
    return plsc.VectorSubcoreMesh(core_axis_name="c", subcore_axis_name="s")


def _make_sc_segsum(nch):
    out_type = jax.ShapeDtypeStruct((NC, NPAD, H), jnp.float32)
    scratch = [
        pltpu.VMEM((nch, CHUNK), jnp.int32),
        pltpu.VMEM((nch, CHUNK), jnp.int32),
        pltpu.VMEM((CHUNK, H), jnp.float32),
        pltpu.VMEM_SHARED((NPAD, H), jnp.float32),
    ]

    def body(table, src_idx, dst_idx, zeros_w, out, isrc, idst, buf, acc):
        c = lax.axis_index("c")
        s = lax.axis_index("s")
        wid = c * NS + s

        pltpu.sync_copy(src_idx.at[wid], isrc)
        pltpu.sync_copy(dst_idx.at[wid], idst)
        pltpu.sync_copy(zeros_w, buf)
        for k in range(_CHPT):
            pltpu.sync_copy(buf, acc.at[pl.ds((s * _CHPT + k) * CHUNK, CHUNK)])
        plsc.subcore_barrier()

        def step(j, carry):
            pltpu.sync_copy(table.at[isrc.at[j]], buf)
            pltpu.sync_copy(buf, acc.at[idst.at[j]], add=True)
            return carry

        lax.fori_loop(0, nch, step, 0)
        plsc.subcore_barrier()

        pltpu.sync_copy(acc.at[pl.ds(s * _ROWS, _ROWS)],
                        out.at[c].at[pl.ds(s * _ROWS, _ROWS)])

    return pl.kernel(body, out_type=out_type, mesh=_sc_mesh(),
                     scratch_types=scratch)


def _make_sc_degree(nch):
    out_type = jax.ShapeDtypeStruct((NC, NPAD, H), jnp.float32)
    scratch = [
        pltpu.VMEM((nch, CHUNK), jnp.int32),
        pltpu.VMEM((2, CHUNK, H), jnp.float32),
        pltpu.VMEM_SHARED((NPAD, H), jnp.float32),
    ]

    def body(dst_idx, zo_in, out, idst, buf, dacc):
        c = lax.axis_index("c")
        s = lax.axis_index("s")
        wid = c * NS + s

        pltpu.sync_copy(dst_idx.at[wid], idst)
        pltpu.sync_copy(zo_in, buf)
        for k in range(_CHPT):
            pltpu.sync_copy(buf.at[0],
                            dacc.at[pl.ds((s * _CHPT + k) * CHUNK, CHUNK)])
        plsc.subcore_barrier()

        def step(j, carry):
            pltpu.sync_copy(buf.at[1], dacc.at[idst.at[j]], add=True)
            return carry

        lax.fori_loop(0, nch, step, 0)
        plsc.subcore_barrier()

        pltpu.sync_copy(dacc.at[pl.ds(s * _ROWS, _ROWS)],
                        out.at[c].at[pl.ds(s * _ROWS, _ROWS)])

    return pl.kernel(body, out_type=out_type, mesh=_sc_mesh(),
                     scratch_types=scratch)


def _dot_t(a, w):
    return lax.dot_general(a, w, (((1,), (1,)), ((), ())),
                           preferred_element_type=jnp.float32)


def _layer_tail(pre, g, be):
    r = jnp.maximum(pre, 0.0)
    mu = jnp.mean(r, axis=-1, keepdims=True)
    var = jnp.mean((r - mu) ** 2, axis=-1, keepdims=True)
    return (r - mu) * lax.rsqrt(var + 1e-5) * g + be


def _tc_first_body(x, wl, y):
    y[...] = _dot_t(x[...], wl[...])


def _tc_mid1_body(x, s, deg, wr, b, g, be, wlnext, h_out, y_out, inv_out):
    inv = 1.0 / jnp.maximum(deg[0] + deg[1], 1.0)
    pre = (s[0] + s[1]) * inv + _dot_t(x[...], wr[...]) + b[...]
    hval = _layer_tail(pre, g[...], be[...])
    h_out[...] = hval
    y_out[...] = _dot_t(hval, wlnext[...])
    inv_out[...] = inv


def _tc_mid2_body(hprev, s, inv, wr, b, g, be, h_out):
    pre = (s[0] + s[1]) * inv[...] + _dot_t(hprev[...], wr[...]) + b[...]
    h_out[...] = _layer_tail(pre, g[...], be[...])


def _tc_last_body(hprev, s, inv, wl, wr, b, out):
    agg = (s[0] + s[1]) * inv[...]
    out[...] = _dot_t(agg, wl[...]) + _dot_t(hprev[...], wr[...]) + b[...]


def _row_spec(w):
    return pl.BlockSpec((BN, w), lambda i: (i, 0))


def _pair_spec(w):
    return pl.BlockSpec((NC, BN, w), lambda i: (0, i, 0))


def _full_spec(r, c):
    return pl.BlockSpec((r, c), lambda i: (0, 0))


def _f32(*shape):
    return jax.ShapeDtypeStruct(shape, jnp.float32)


def _tc_first(x, wl):
    return pl.pallas_call(
        _tc_first_body,
        grid=(NUM_NODES // BN,),
        in_specs=[_row_spec(H), _full_spec(H, H)],
        out_specs=_row_spec(H),
        out_shape=_f32(NUM_NODES, H),
    )(x, wl)


def _tc_mid1(x, s, deg, wr, b, g, be, wlnext):
    return pl.pallas_call(
        _tc_mid1_body,
        grid=(NUM_NODES // BN,),
        in_specs=[_row_spec(H), _pair_spec(H), _pair_spec(H),
                  _full_spec(H, H), _full_spec(1, H), _full_spec(1, H),
                  _full_spec(1, H), _full_spec(H, H)],
        out_specs=[_row_spec(H), _row_spec(H), _row_spec(H)],
        out_shape=[_f32(NUM_NODES, H), _f32(NUM_NODES, H),
                   _f32(NUM_NODES, H)],
    )(x, s, deg, wr, b, g, be, wlnext)


def _tc_mid2(hprev, s, inv, wr, b, g, be):
    return pl.pallas_call(
        _tc_mid2_body,
        grid=(NUM_NODES // BN,),
        in_specs=[_row_spec(H), _pair_spec(H), _row_spec(H),
                  _full_spec(H, H), _full_spec(1, H), _full_spec(1, H),
                  _full_spec(1, H)],
        out_specs=_row_spec(H),
        out_shape=_f32(NUM_NODES, H),
    )(hprev, s, inv, wr, b, g, be)


def _tc_last(hprev, s, inv, wl, wr, b):
    return pl.pallas_call(
        _tc_last_body,
        grid=(NUM_NODES // BN,),
        in_specs=[_row_spec(H), _pair_spec(H), _row_spec(H),
                  _full_spec(64, H), _full_spec(64, H), _full_spec(1, 64)],
        out_specs=_row_spec(64),
        out_shape=_f32(NUM_NODES, 64),
    )(hprev, s, inv, wl, wr, b)


@jax.jit
def kernel(x, edge_index, Wl1, Wr1, b1, g1, be1, Wl2, Wr2, b2, g2, be2,
           Wl3, Wr3, b3):
    E = edge_index.shape[1]
    e_pw = -(-E // NW)
    nch = -(-e_pw // CHUNK)
    pad = NW * nch * CHUNK - E

    src = jnp.concatenate(
        [edge_index[0], jnp.zeros((pad,), jnp.int32)]).reshape(NW, nch, CHUNK)
    dst = jnp.concatenate(
        [edge_index[1],
         jnp.full((pad,), NUM_NODES, jnp.int32)]).reshape(NW, nch, CHUNK)

    zeros_w = jnp.zeros((CHUNK, H), jnp.float32)
    zo = jnp.stack([zeros_w, jnp.ones((CHUNK, H), jnp.float32)])

    b1r, g1r, be1r = b1.reshape(1, -1), g1.reshape(1, -1), be1.reshape(1, -1)
    b2r, g2r, be2r = b2.reshape(1, -1), g2.reshape(1, -1), be2.reshape(1, -1)
    b3r = b3.reshape(1, -1)

    sc_segsum = _make_sc_segsum(nch)
    sc_degree = _make_sc_degree(nch)

    degraw = sc_degree(dst, zo)
    y1 = _tc_first(x, Wl1)
    s1 = sc_segsum(y1, src, dst, zeros_w)
    h1, y2, inv = _tc_mid1(x, s1, degraw, Wr1, b1r, g1r, be1r, Wl2)
    s2 = sc_segsum(y2, src, dst, zeros_w)
    h2 = _tc_mid2(h1, s2, inv, Wr2, b2r, g2r, be2r)
    s3 = sc_segsum(h2, src, dst, zeros_w)
    out = _tc_last(h2, s3, inv, Wl3, Wr3, b3r)
    return out
